# baseline (speedup 1.0000x reference)
import functools

import jax
import jax.numpy as jnp
from jax import lax
from jax.experimental import pallas as pl
from jax.experimental.pallas import tpu as pltpu
from jax.experimental.pallas import tpu_sc as plsc

NC = 2
NS = 16
L = 16
H = 128
W = 128
FDIM = 256
BOXC = 16
RC = 32
NK = H // RC
CHUNK = RC * W


def _pair_span(p, box_v):
    bv = box_v[pl.ds(p * BOXC, L)]
    sy1 = bv[1]
    oy1 = bv[5]
    sy2c = bv[9]
    oy2c = bv[11]
    valid = ((bv[3] - sy1 >= 5) & (bv[2] - bv[0] >= 5)
             & (bv[7] - oy1 >= 5) & (bv[6] - bv[4] >= 5))
    y0 = jnp.clip(jnp.minimum(sy1, oy1), 0, H)
    y1 = jnp.clip(jnp.maximum(sy2c, oy2c), y0, H)
    y1 = jnp.where(valid, y1, y0)
    k0 = y0 >> 5
    k1 = (y1 + (RC - 1)) >> 5
    k1 = jnp.where(y1 > y0, k1, k0)
    return k0, k1


def _fill_pair(p, k0, k1, rel_v, box_v, ratio_v, canvas):
    f32 = jnp.float32
    i32 = jnp.int32

    bv = box_v[pl.ds(p * BOXC, L)]
    sx1 = bv[0]
    sy1 = bv[1]
    sy2 = bv[3]
    ox1 = bv[4]
    oy1 = bv[5]
    oy2 = bv[7]
    sx2c = bv[8]
    sy2c = bv[9]
    ox2c = bv[10]
    oy2c = bv[11]

    sh = sy2 - sy1
    sw = bv[2] - sx1
    oh = oy2 - oy1
    ow = bv[6] - ox1
    valid = (sh >= 5) & (sw >= 5) & (oh >= 5) & (ow >= 5)

    rv = ratio_v[pl.ds(p * BOXC, L)]
    ratio_s = rv[0]
    ratio_o = rv[1]

    lane = lax.iota(i32, L)
    rel_base = jnp.full((L,), p * FDIM, i32)

    cs_regs = []
    co_regs = []
    for c in range(W // L):
        x = lane + (c * L)
        cs_regs.append(jnp.where((x >= sx1) & (x < sx2c), f32(1), f32(0)))
        co_regs.append(jnp.where((x >= ox1) & (x < ox2c), f32(1), f32(0)))

    def chunk_body(k, carry):
        y = lane + k * L

        t_s = (y - sy1).astype(f32) * ratio_s
        rs = jnp.clip(t_s, f32(0), f32(FDIM - 1))
        g_s = plsc.load_gather(rel_v, [rel_base + rs.astype(i32)])
        m_s = valid & (y >= sy1) & (y < sy2c)
        vs_c = jnp.where(m_s, g_s, f32(0))

        t_o = (y - oy1).astype(f32) * ratio_o
        ro = jnp.clip(t_o, f32(0), f32(FDIM - 1))
        g_o = plsc.load_gather(rel_v, [rel_base + ro.astype(i32)])
        m_o = valid & (y >= oy1) & (y < oy2c)
        vo_c = jnp.where(m_o, g_o, f32(0))

        row_base = k * (L * W)
        for j in range(L):
            vs = vs_c[j]
            vo = vo_c[j]
            for c in range(W // L):
                canvas[pl.ds(row_base + j * W + c * L, L)] = (
                    vs * cs_regs[c] + vo * co_regs[c])
        return carry

    lax.fori_loop(k0 * (RC // L), k1 * (RC // L), chunk_body, 0)


def _do_pair(p, base, rel_v, box_v, ratio_v, canvas, zero_v,
             sem, semz, out_hbm, kc_prev):
    def wait_strip(_, c):
        pltpu.make_async_copy(
            canvas.at[pl.ds(0, CHUNK)], out_hbm.at[pl.ds(0, CHUNK)],
            sem).wait()
        return c

    def wait_zero(_, c):
        pltpu.make_async_copy(
            zero_v.at[pl.ds(0, CHUNK)], out_hbm.at[pl.ds(0, CHUNK)],
            semz).wait()
        return c

    lax.fori_loop(0, jnp.maximum(kc_prev, 0), wait_strip, 0)
    lax.fori_loop(0, jnp.where(kc_prev < 0, 0, NK - kc_prev), wait_zero, 0)

    k0, k1 = _pair_span(p, box_v)
    out0 = (base + p) * (H * W)
    nc = k1 - k0
    nsuf = NK - k1

    for n in range(1, NK + 1):
        @pl.when(k0 == n)
        def _(n=n):
            pltpu.async_copy(
                zero_v.at[pl.ds(0, n * CHUNK)],
                out_hbm.at[pl.ds(out0, n * CHUNK)], semz)

        @pl.when(nsuf == n)
        def _(n=n):
            pltpu.async_copy(
                zero_v.at[pl.ds(0, n * CHUNK)],
                out_hbm.at[pl.ds(out0 + k1 * CHUNK, n * CHUNK)], semz)

    _fill_pair(p, k0, k1, rel_v, box_v, ratio_v, canvas)

    for n in range(1, NK + 1):
        @pl.when(nc == n)
        def _(n=n):
            pltpu.async_copy(
                canvas.at[pl.ds(k0 * CHUNK, n * CHUNK)],
                out_hbm.at[pl.ds(out0 + k0 * CHUNK, n * CHUNK)], sem)
    return nc


def _sc_kernel(ppw,
               rel_hbm, bbox_hbm, out_hbm,
               rel_v, bboxr_v, box_v, ratio_v, canvas0, canvas1, zero_v,
               sem0, sem1, semz, semin):
    wid = lax.axis_index("s") * NC + lax.axis_index("c")
    base = wid * ppw

    c_rel = pltpu.async_copy(
        rel_hbm.at[pl.ds(base * FDIM, ppw * FDIM)], rel_v, semin)
    c_box = pltpu.async_copy(
        bbox_hbm.at[pl.ds(base * 8, ppw * 8)], bboxr_v, semin)

    zreg = jnp.zeros((L,), jnp.float32)

    def zinit(i, c):
        zero_v[pl.ds(i * L, L)] = zreg
        return c

    lax.fori_loop(0, (NK * CHUNK) // L, zinit, 0)
    c_rel.wait()
    c_box.wait()

    f32 = jnp.float32
    i32 = jnp.int32
    pl16 = lax.iota(i32, L)

    def gf(f):
        return plsc.load_gather(bboxr_v, [pl16 * 8 + f])

    sx1 = gf(0) >> 1
    sy1 = gf(1) >> 1
    sx2 = gf(2) >> 1
    sy2 = gf(3) >> 1
    ox1 = gf(4) >> 1
    oy1 = gf(5) >> 1
    ox2 = gf(6) >> 1
    oy2 = gf(7) >> 1
    fields = [sx1, sy1, sx2, sy2, ox1, oy1, ox2, oy2,
              jnp.minimum(sx2, W), jnp.minimum(sy2, H),
              jnp.minimum(ox2, W), jnp.minimum(oy2, H)]
    fdim = f32(FDIM)
    ratio_s = fdim / jnp.maximum(sy2 - sy1, 1).astype(f32)
    ratio_o = fdim / jnp.maximum(oy2 - oy1, 1).astype(f32)

    bmeta = pl16 * BOXC
    for f, v in enumerate(fields):
        plsc.store_scatter(box_v, [bmeta + f], v)
    plsc.store_scatter(ratio_v, [bmeta], ratio_s)
    plsc.store_scatter(ratio_v, [bmeta + 1], ratio_o)

    def pair_step(j, carry):
        kc0_prev, kc1_prev = carry
        kc0 = _do_pair(2 * j, base, rel_v, box_v, ratio_v, canvas0, zero_v,
                       sem0, semz, out_hbm, kc0_prev)
        kc1 = _do_pair(2 * j + 1, base, rel_v, box_v, ratio_v, canvas1,
                       zero_v, sem1, semz, out_hbm, kc1_prev)
        return kc0, kc1

    kc0, kc1 = lax.fori_loop(0, ppw // 2, pair_step,
                             (jnp.int32(-1), jnp.int32(-1)))

    def wait0(_, c):
        pltpu.make_async_copy(canvas0.at[pl.ds(0, CHUNK)],
                              out_hbm.at[pl.ds(0, CHUNK)], sem0).wait()
        return c

    def wait1(_, c):
        pltpu.make_async_copy(canvas1.at[pl.ds(0, CHUNK)],
                              out_hbm.at[pl.ds(0, CHUNK)], sem1).wait()
        return c

    def waitz(_, c):
        pltpu.make_async_copy(zero_v.at[pl.ds(0, CHUNK)],
                              out_hbm.at[pl.ds(0, CHUNK)], semz).wait()
        return c

    lax.fori_loop(0, kc0, wait0, 0)
    lax.fori_loop(0, kc1, wait1, 0)
    lax.fori_loop(0, 2 * NK - kc0 - kc1, waitz, 0)


def kernel(rel_features, bbox, size):
    del size
    B, N, Fdim = rel_features.shape
    pairs = B * N
    ppw = pairs // (NC * NS)

    rel2 = rel_features.reshape(pairs * Fdim)
    bbox2 = bbox.astype(jnp.int32).reshape(pairs * 8)

    mesh = plsc.VectorSubcoreMesh(core_axis_name="c", subcore_axis_name="s",
                                  num_cores=NC, num_subcores=NS)
    body = functools.partial(_sc_kernel, ppw)
    out = pl.kernel(
        body,
        out_type=jax.ShapeDtypeStruct((pairs * H * W,), jnp.float32),
        mesh=mesh,
        compiler_params=pltpu.CompilerParams(needs_layout_passes=False),
        scratch_types=[
            pltpu.VMEM((ppw * FDIM,), jnp.float32),
            pltpu.VMEM((ppw * 8,), jnp.int32),
            pltpu.VMEM((ppw * BOXC,), jnp.int32),
            pltpu.VMEM((ppw * BOXC,), jnp.float32),
            pltpu.VMEM((H * W,), jnp.float32),
            pltpu.VMEM((H * W,), jnp.float32),
            pltpu.VMEM((NK * CHUNK,), jnp.float32),
            pltpu.SemaphoreType.DMA,
            pltpu.SemaphoreType.DMA,
            pltpu.SemaphoreType.DMA,
            pltpu.SemaphoreType.DMA,
        ],
    )(rel2, bbox2)
    return out.reshape(B, N, H, W)


# speedup vs baseline: 1.0264x; 1.0264x over previous
import functools

import jax
import jax.numpy as jnp
from jax import lax
from jax.experimental import pallas as pl
from jax.experimental.pallas import tpu as pltpu
from jax.experimental.pallas import tpu_sc as plsc

NC = 2
NS = 16
L = 16
H = 128
W = 128
FDIM = 256
BOXC = 16
RC = 32
NK = H // RC
CHUNK = RC * W


def _fill_pair(p, rel_v, box_v, ratio_v, canvas):
    f32 = jnp.float32
    i32 = jnp.int32

    bv = box_v[pl.ds(p * BOXC, L)]
    sx1 = bv[0]
    sy1 = bv[1]
    sy2 = bv[3]
    ox1 = bv[4]
    oy1 = bv[5]
    oy2 = bv[7]
    sx2c = bv[8]
    sy2c = bv[9]
    ox2c = bv[10]
    oy2c = bv[11]

    sh = sy2 - sy1
    sw = bv[2] - sx1
    oh = oy2 - oy1
    ow = bv[6] - ox1
    valid = (sh >= 5) & (sw >= 5) & (oh >= 5) & (ow >= 5)

    rv = ratio_v[pl.ds(p * BOXC, L)]
    ratio_s = rv[0]
    ratio_o = rv[1]

    y0 = jnp.clip(jnp.minimum(sy1, oy1), 0, H)
    y1 = jnp.clip(jnp.maximum(sy2c, oy2c), y0, H)
    y1 = jnp.where(valid, y1, y0)
    k0 = y0 >> 5
    k1 = (y1 + (RC - 1)) >> 5
    k1 = jnp.where(y1 > y0, k1, k0)

    lane = lax.iota(i32, L)
    rel_base = jnp.full((L,), p * FDIM, i32)

    cs_regs = []
    co_regs = []
    for c in range(W // L):
        x = lane + (c * L)
        cs_regs.append(jnp.where((x >= sx1) & (x < sx2c), f32(1), f32(0)))
        co_regs.append(jnp.where((x >= ox1) & (x < ox2c), f32(1), f32(0)))

    def chunk_body(k, carry):
        y = lane + k * L

        t_s = (y - sy1).astype(f32) * ratio_s
        rs = jnp.clip(t_s, f32(0), f32(FDIM - 1))
        g_s = plsc.load_gather(rel_v, [rel_base + rs.astype(i32)])
        m_s = valid & (y >= sy1) & (y < sy2c)
        vs_c = jnp.where(m_s, g_s, f32(0))

        t_o = (y - oy1).astype(f32) * ratio_o
        ro = jnp.clip(t_o, f32(0), f32(FDIM - 1))
        g_o = plsc.load_gather(rel_v, [rel_base + ro.astype(i32)])
        m_o = valid & (y >= oy1) & (y < oy2c)
        vo_c = jnp.where(m_o, g_o, f32(0))

        row_base = k * (L * W)
        for j in range(L):
            vs = vs_c[j]
            vo = vo_c[j]
            for c in range(W // L):
                canvas[pl.ds(row_base + j * W + c * L, L)] = (
                    vs * cs_regs[c] + vo * co_regs[c])
        return carry

    lax.fori_loop(k0 * (RC // L), k1 * (RC // L), chunk_body, 0)
    return k0, k1


def _do_pair(p, base, rel_v, box_v, ratio_v, canvas, zero_v,
             sem, semz, out_hbm, kc_prev):
    def wait_strip(_, c):
        pltpu.make_async_copy(
            canvas.at[pl.ds(0, CHUNK)], out_hbm.at[pl.ds(0, CHUNK)],
            sem).wait()
        return c

    def wait_zero(_, c):
        pltpu.make_async_copy(
            zero_v.at[pl.ds(0, CHUNK)], out_hbm.at[pl.ds(0, CHUNK)],
            semz).wait()
        return c

    lax.fori_loop(0, jnp.maximum(kc_prev, 0), wait_strip, 0)
    lax.fori_loop(0, jnp.where(kc_prev < 0, 0, NK - kc_prev), wait_zero, 0)

    k0, k1 = _fill_pair(p, rel_v, box_v, ratio_v, canvas)

    out0 = (base + p) * (H * W)

    nc = k1 - k0
    nsuf = NK - k1
    for n in range(1, NK + 1):
        @pl.when(nc == n)
        def _(n=n):
            pltpu.async_copy(
                canvas.at[pl.ds(k0 * CHUNK, n * CHUNK)],
                out_hbm.at[pl.ds(out0 + k0 * CHUNK, n * CHUNK)], sem)

        @pl.when(k0 == n)
        def _(n=n):
            pltpu.async_copy(
                zero_v.at[pl.ds(0, n * CHUNK)],
                out_hbm.at[pl.ds(out0, n * CHUNK)], semz)

        @pl.when(nsuf == n)
        def _(n=n):
            pltpu.async_copy(
                zero_v.at[pl.ds(0, n * CHUNK)],
                out_hbm.at[pl.ds(out0 + k1 * CHUNK, n * CHUNK)], semz)
    return nc


def _sc_kernel(ppw,
               rel_hbm, bbox_hbm, out_hbm,
               rel_v, bboxr_v, box_v, ratio_v, canvas0, canvas1, zero_v,
               sem0, sem1, semz, semin):
    wid = lax.axis_index("s") * NC + lax.axis_index("c")
    base = wid * ppw

    c_rel = pltpu.async_copy(
        rel_hbm.at[pl.ds(base * FDIM, ppw * FDIM)], rel_v, semin)
    c_box = pltpu.async_copy(
        bbox_hbm.at[pl.ds(base * 8, ppw * 8)], bboxr_v, semin)

    zreg = jnp.zeros((L,), jnp.float32)

    def zinit(i, c):
        zero_v[pl.ds(i * L, L)] = zreg
        return c

    lax.fori_loop(0, (NK * CHUNK) // L, zinit, 0)
    c_rel.wait()
    c_box.wait()

    f32 = jnp.float32
    i32 = jnp.int32
    pl16 = lax.iota(i32, L)

    def gf(f):
        return plsc.load_gather(bboxr_v, [pl16 * 8 + f])

    sx1 = gf(0) >> 1
    sy1 = gf(1) >> 1
    sx2 = gf(2) >> 1
    sy2 = gf(3) >> 1
    ox1 = gf(4) >> 1
    oy1 = gf(5) >> 1
    ox2 = gf(6) >> 1
    oy2 = gf(7) >> 1
    fields = [sx1, sy1, sx2, sy2, ox1, oy1, ox2, oy2,
              jnp.minimum(sx2, W), jnp.minimum(sy2, H),
              jnp.minimum(ox2, W), jnp.minimum(oy2, H)]
    fdim = f32(FDIM)
    ratio_s = fdim / jnp.maximum(sy2 - sy1, 1).astype(f32)
    ratio_o = fdim / jnp.maximum(oy2 - oy1, 1).astype(f32)

    bmeta = pl16 * BOXC
    for f, v in enumerate(fields):
        plsc.store_scatter(box_v, [bmeta + f], v)
    plsc.store_scatter(ratio_v, [bmeta], ratio_s)
    plsc.store_scatter(ratio_v, [bmeta + 1], ratio_o)

    def pair_step(j, carry):
        kc0_prev, kc1_prev = carry
        kc0 = _do_pair(2 * j, base, rel_v, box_v, ratio_v, canvas0, zero_v,
                       sem0, semz, out_hbm, kc0_prev)
        kc1 = _do_pair(2 * j + 1, base, rel_v, box_v, ratio_v, canvas1,
                       zero_v, sem1, semz, out_hbm, kc1_prev)
        return kc0, kc1

    kc0, kc1 = lax.fori_loop(0, ppw // 2, pair_step,
                             (jnp.int32(-1), jnp.int32(-1)))

    def wait0(_, c):
        pltpu.make_async_copy(canvas0.at[pl.ds(0, CHUNK)],
                              out_hbm.at[pl.ds(0, CHUNK)], sem0).wait()
        return c

    def wait1(_, c):
        pltpu.make_async_copy(canvas1.at[pl.ds(0, CHUNK)],
                              out_hbm.at[pl.ds(0, CHUNK)], sem1).wait()
        return c

    def waitz(_, c):
        pltpu.make_async_copy(zero_v.at[pl.ds(0, CHUNK)],
                              out_hbm.at[pl.ds(0, CHUNK)], semz).wait()
        return c

    lax.fori_loop(0, kc0, wait0, 0)
    lax.fori_loop(0, kc1, wait1, 0)
    lax.fori_loop(0, 2 * NK - kc0 - kc1, waitz, 0)


def kernel(rel_features, bbox, size):
    del size
    B, N, Fdim = rel_features.shape
    pairs = B * N
    ppw = pairs // (NC * NS)

    rel2 = rel_features.reshape(pairs * Fdim)
    bbox2 = bbox.astype(jnp.int32).reshape(pairs * 8)

    mesh = plsc.VectorSubcoreMesh(core_axis_name="c", subcore_axis_name="s",
                                  num_cores=NC, num_subcores=NS)
    body = functools.partial(_sc_kernel, ppw)
    out = pl.kernel(
        body,
        out_type=jax.ShapeDtypeStruct((pairs * H * W,), jnp.float32),
        mesh=mesh,
        compiler_params=pltpu.CompilerParams(needs_layout_passes=False),
        scratch_types=[
            pltpu.VMEM((ppw * FDIM,), jnp.float32),
            pltpu.VMEM((ppw * 8,), jnp.int32),
            pltpu.VMEM((ppw * BOXC,), jnp.int32),
            pltpu.VMEM((ppw * BOXC,), jnp.float32),
            pltpu.VMEM((H * W,), jnp.float32),
            pltpu.VMEM((H * W,), jnp.float32),
            pltpu.VMEM((NK * CHUNK,), jnp.float32),
            pltpu.SemaphoreType.DMA,
            pltpu.SemaphoreType.DMA,
            pltpu.SemaphoreType.DMA,
            pltpu.SemaphoreType.DMA,
        ],
    )(rel2, bbox2)
    return out.reshape(B, N, H, W)
